# baseline (speedup 1.0000x reference)
import functools

import jax
import jax.numpy as jnp
from jax import lax
from jax.experimental import pallas as pl
from jax.experimental.pallas import tpu as pltpu
from jax.experimental.pallas import tpu_sc as plsc

NC = 2
NS = 16
NW = NC * NS
D = 64
CH = 128


def _sc_dots(u_idx, v_idx, W, n_pairs):
    pw = n_pairs // NW
    nch = pw // CH

    mesh = plsc.VectorSubcoreMesh(core_axis_name="c", subcore_axis_name="s")

    @functools.partial(
        pl.kernel,
        out_type=jax.ShapeDtypeStruct((NW, nch, CH), jnp.float32),
        mesh=mesh,
        scratch_types=[
            pltpu.VMEM((nch, CH), jnp.int32),
            pltpu.VMEM((nch, CH), jnp.int32),
            pltpu.VMEM((CH, D), jnp.float32),
            pltpu.VMEM((CH, D), jnp.float32),
            pltpu.VMEM((nch, CH), jnp.float32),
            pltpu.SemaphoreType.DMA,
            pltpu.SemaphoreType.DMA,
        ],
        compiler_params=pltpu.CompilerParams(
            needs_layout_passes=False, use_tc_tiling_on_sc=False),
    )
    def run(u_hbm, v_hbm, w_hbm, out_hbm, uix, vix, ru, rv, dots, s1, s2):
        w = lax.axis_index("s") * NC + lax.axis_index("c")
        pltpu.sync_copy(u_hbm.at[w], uix)
        pltpu.sync_copy(v_hbm.at[w], vix)

        def chunk(c, carry):
            cu = pltpu.async_copy(w_hbm.at[uix.at[c]], ru, s1)
            cv = pltpu.async_copy(w_hbm.at[vix.at[c]], rv, s2)
            cu.wait()
            cv.wait()

            def group(g, carry2):
                rowv = lax.iota(jnp.int32, 16) + g * 16
                acc = jnp.zeros((16,), jnp.float32)
                for j in range(D):
                    colv = jnp.full((16,), j, jnp.int32)
                    acc = acc + (plsc.load_gather(ru, [rowv, colv])
                                 * plsc.load_gather(rv, [rowv, colv]))
                dots[c, pl.ds(g * 16, 16)] = acc
                return carry2

            lax.fori_loop(0, CH // 16, group, 0, unroll=False)
            return carry

        lax.fori_loop(0, nch, chunk, 0, unroll=False)
        pltpu.sync_copy(dots, out_hbm.at[w])

    return run(u_idx.reshape(NW, nch, CH), v_idx.reshape(NW, nch, CH), W)


def _tc_loss(dots, ys, n_pos, n_pairs):
    rows = n_pairs // 128
    pos_rows = n_pos // 128
    scale = 1.0 / float(n_pairs)

    def body(dots_ref, ys_ref, o_ref):
        x = dots_ref[...]
        y = ys_ref[...]
        alpha = jnp.log(y * y + 1.0) + 1.0
        pos = x[0:pos_rows, :]
        neg = x[pos_rows:rows, :]
        ls_pos = jnp.minimum(pos, 0.0) - jnp.log(1.0 + jnp.exp(-jnp.abs(pos)))
        ls_neg = jnp.minimum(-neg, 0.0) - jnp.log(1.0 + jnp.exp(-jnp.abs(neg)))
        total = jnp.sum(-alpha * ls_pos) + jnp.sum(-ls_neg)
        o_ref[0, 0] = total * scale

    out = pl.pallas_call(
        body,
        out_shape=jax.ShapeDtypeStruct((1, 1), jnp.float32),
        out_specs=pl.BlockSpec(memory_space=pltpu.SMEM),
    )(dots.reshape(rows, 128), ys.reshape(n_pos // 128, 128))
    return out[0, 0]


def kernel(pos_idxs, ys, neg_idxs, W, num_neg):
    n_pos = pos_idxs.shape[1]
    n_pairs = n_pos + neg_idxs.shape[1]
    u_idx = jnp.concatenate([pos_idxs[0], neg_idxs[0]]).astype(jnp.int32)
    v_idx = jnp.concatenate([pos_idxs[1], neg_idxs[1]]).astype(jnp.int32)
    dots = _sc_dots(u_idx, v_idx, W, n_pairs)
    return _tc_loss(dots.reshape(n_pairs), ys, n_pos, n_pairs)


# speedup vs baseline: 1.0476x; 1.0476x over previous
import functools

import jax
import jax.numpy as jnp
from jax import lax
from jax.experimental import pallas as pl
from jax.experimental.pallas import tpu as pltpu
from jax.experimental.pallas import tpu_sc as plsc

NC = 2
NS = 16
NW = NC * NS
D = 64
CH = 128


def _sc_dots(u_idx, v_idx, W, n_pairs):
    pw = n_pairs // NW
    nch = pw // CH

    mesh = plsc.VectorSubcoreMesh(core_axis_name="c", subcore_axis_name="s")

    @functools.partial(
        pl.kernel,
        out_type=jax.ShapeDtypeStruct((NW, nch, CH), jnp.float32),
        mesh=mesh,
        scratch_types=[
            pltpu.VMEM((nch, CH), jnp.int32),
            pltpu.VMEM((nch, CH), jnp.int32),
            pltpu.VMEM((CH, D), jnp.float32),
            pltpu.VMEM((CH, D), jnp.float32),
            pltpu.VMEM((CH, D), jnp.float32),
            pltpu.VMEM((CH, D), jnp.float32),
            pltpu.VMEM((nch, CH), jnp.float32),
            pltpu.SemaphoreType.DMA,
            pltpu.SemaphoreType.DMA,
            pltpu.SemaphoreType.DMA,
            pltpu.SemaphoreType.DMA,
        ],
        compiler_params=pltpu.CompilerParams(
            needs_layout_passes=False, use_tc_tiling_on_sc=False),
    )
    def run(u_hbm, v_hbm, w_hbm, out_hbm, uix, vix,
            ru0, rv0, ru1, rv1, dots, su0, sv0, su1, sv1):
        w = lax.axis_index("s") * NC + lax.axis_index("c")
        pltpu.sync_copy(u_hbm.at[w], uix)
        pltpu.sync_copy(v_hbm.at[w], vix)

        def fetch(c, ru, rv, su, sv):
            pltpu.async_copy(w_hbm.at[uix.at[c]], ru, su)
            pltpu.async_copy(w_hbm.at[vix.at[c]], rv, sv)

        def drain(ru, rv, su, sv):
            pltpu.make_async_copy(w_hbm.at[uix.at[0]], ru, su).wait()
            pltpu.make_async_copy(w_hbm.at[vix.at[0]], rv, sv).wait()

        def dot_chunk(c, ru, rv):
            def group(g, carry2):
                rowv = lax.iota(jnp.int32, 16) + g * 16
                acc = jnp.zeros((16,), jnp.float32)
                for j in range(D):
                    colv = jnp.full((16,), j, jnp.int32)
                    acc = acc + (plsc.load_gather(ru, [rowv, colv])
                                 * plsc.load_gather(rv, [rowv, colv]))
                dots[c, pl.ds(g * 16, 16)] = acc
                return carry2

            lax.fori_loop(0, CH // 16, group, 0, unroll=False)

        fetch(0, ru0, rv0, su0, sv0)
        fetch(1, ru1, rv1, su1, sv1)

        def pair(i, carry):
            c0 = 2 * i
            drain(ru0, rv0, su0, sv0)
            dot_chunk(c0, ru0, rv0)
            cn0 = jnp.minimum(c0 + 2, nch - 2)
            fetch(cn0, ru0, rv0, su0, sv0)
            c1 = c0 + 1
            drain(ru1, rv1, su1, sv1)
            dot_chunk(c1, ru1, rv1)
            cn1 = jnp.minimum(c1 + 2, nch - 1)
            fetch(cn1, ru1, rv1, su1, sv1)
            return carry

        lax.fori_loop(0, nch // 2, pair, 0, unroll=False)
        drain(ru0, rv0, su0, sv0)
        drain(ru1, rv1, su1, sv1)
        pltpu.sync_copy(dots, out_hbm.at[w])

    return run(u_idx.reshape(NW, nch, CH), v_idx.reshape(NW, nch, CH), W)


def _tc_loss(dots, ys, n_pos, n_pairs):
    rows = n_pairs // 128
    pos_rows = n_pos // 128
    scale = 1.0 / float(n_pairs)

    def body(dots_ref, ys_ref, o_ref):
        x = dots_ref[...]
        y = ys_ref[...]
        alpha = jnp.log(y * y + 1.0) + 1.0
        pos = x[0:pos_rows, :]
        neg = x[pos_rows:rows, :]
        ls_pos = jnp.minimum(pos, 0.0) - jnp.log(1.0 + jnp.exp(-jnp.abs(pos)))
        ls_neg = jnp.minimum(-neg, 0.0) - jnp.log(1.0 + jnp.exp(-jnp.abs(neg)))
        total = jnp.sum(-alpha * ls_pos) + jnp.sum(-ls_neg)
        o_ref[0, 0] = total * scale

    out = pl.pallas_call(
        body,
        out_shape=jax.ShapeDtypeStruct((1, 1), jnp.float32),
        out_specs=pl.BlockSpec(memory_space=pltpu.SMEM),
    )(dots.reshape(rows, 128), ys.reshape(n_pos // 128, 128))
    return out[0, 0]


def kernel(pos_idxs, ys, neg_idxs, W, num_neg):
    n_pos = pos_idxs.shape[1]
    n_pairs = n_pos + neg_idxs.shape[1]
    u_idx = jnp.concatenate([pos_idxs[0], neg_idxs[0]]).astype(jnp.int32)
    v_idx = jnp.concatenate([pos_idxs[1], neg_idxs[1]]).astype(jnp.int32)
    dots = _sc_dots(u_idx, v_idx, W, n_pairs)
    return _tc_loss(dots.reshape(n_pairs), ys, n_pos, n_pairs)


# speedup vs baseline: 1.3682x; 1.3061x over previous
import functools

import jax
import jax.numpy as jnp
from jax import lax
from jax.experimental import pallas as pl
from jax.experimental.pallas import tpu as pltpu
from jax.experimental.pallas import tpu_sc as plsc

NC = 2
NS = 16
NW = NC * NS
D = 64
CH = 128


def _sc_dots(u_idx, v_idx, W, n_pairs):
    pw = n_pairs // NW
    nch = pw // CH

    mesh = plsc.VectorSubcoreMesh(core_axis_name="c", subcore_axis_name="s")

    @functools.partial(
        pl.kernel,
        out_type=jax.ShapeDtypeStruct((NW, nch, CH), jnp.float32),
        mesh=mesh,
        scratch_types=[
            pltpu.VMEM((nch, CH), jnp.int32),
            pltpu.VMEM((nch, CH), jnp.int32),
            pltpu.VMEM((CH, D), jnp.float32),
            pltpu.VMEM((CH, D), jnp.float32),
            pltpu.VMEM((CH, D), jnp.float32),
            pltpu.VMEM((CH, D), jnp.float32),
            pltpu.VMEM((nch, CH), jnp.float32),
            pltpu.SemaphoreType.DMA,
            pltpu.SemaphoreType.DMA,
            pltpu.SemaphoreType.DMA,
            pltpu.SemaphoreType.DMA,
        ],
        compiler_params=pltpu.CompilerParams(
            needs_layout_passes=False, use_tc_tiling_on_sc=True),
    )
    def run(u_hbm, v_hbm, w_hbm, out_hbm, uix, vix,
            ru0, rv0, ru1, rv1, dots, su0, sv0, su1, sv1):
        w = lax.axis_index("s") * NC + lax.axis_index("c")
        pltpu.sync_copy(u_hbm.at[w], uix)
        pltpu.sync_copy(v_hbm.at[w], vix)

        def fetch(c, ru, rv, su, sv):
            def fbody(t, carry):
                iu = uix[c, pl.ds(t * 16, 16)]
                iv = vix[c, pl.ds(t * 16, 16)]
                for q in range(16):
                    p = t * 16 + q
                    pltpu.async_copy(w_hbm.at[iu[q]], ru.at[p], su)
                    pltpu.async_copy(w_hbm.at[iv[q]], rv.at[p], sv)
                return carry

            lax.fori_loop(0, CH // 16, fbody, 0, unroll=False)

        def drain(ru, rv, su, sv):
            pltpu.make_async_copy(w_hbm.at[pl.ds(0, CH)], ru, su).wait()
            pltpu.make_async_copy(w_hbm.at[pl.ds(0, CH)], rv, sv).wait()

        def dot_chunk(c, ru, rv):
            def group(g, carry2):
                rowv = lax.iota(jnp.int32, 16) + g * 16
                accs = [jnp.zeros((16,), jnp.float32) for _ in range(4)]
                for j in range(D):
                    colv = jnp.full((16,), j, jnp.int32)
                    accs[j % 4] = accs[j % 4] + (
                        plsc.load_gather(ru, [rowv, colv])
                        * plsc.load_gather(rv, [rowv, colv]))
                dots[c, pl.ds(g * 16, 16)] = (
                    (accs[0] + accs[1]) + (accs[2] + accs[3]))
                return carry2

            lax.fori_loop(0, CH // 16, group, 0, unroll=False)

        fetch(0, ru0, rv0, su0, sv0)
        fetch(1, ru1, rv1, su1, sv1)

        def pair(i, carry):
            c0 = 2 * i
            drain(ru0, rv0, su0, sv0)
            dot_chunk(c0, ru0, rv0)
            cn0 = jnp.minimum(c0 + 2, nch - 2)
            fetch(cn0, ru0, rv0, su0, sv0)
            c1 = c0 + 1
            drain(ru1, rv1, su1, sv1)
            dot_chunk(c1, ru1, rv1)
            cn1 = jnp.minimum(c1 + 2, nch - 1)
            fetch(cn1, ru1, rv1, su1, sv1)
            return carry

        lax.fori_loop(0, nch // 2, pair, 0, unroll=False)
        drain(ru0, rv0, su0, sv0)
        drain(ru1, rv1, su1, sv1)
        pltpu.sync_copy(dots, out_hbm.at[w])

    return run(u_idx.reshape(NW, nch, CH), v_idx.reshape(NW, nch, CH), W)


def _tc_loss(dots, ys, n_pos, n_pairs):
    rows = n_pairs // 128
    pos_rows = n_pos // 128
    scale = 1.0 / float(n_pairs)

    def body(dots_ref, ys_ref, o_ref):
        x = dots_ref[...]
        y = ys_ref[...]
        alpha = jnp.log(y * y + 1.0) + 1.0
        pos = x[0:pos_rows, :]
        neg = x[pos_rows:rows, :]
        ls_pos = jnp.minimum(pos, 0.0) - jnp.log(1.0 + jnp.exp(-jnp.abs(pos)))
        ls_neg = jnp.minimum(-neg, 0.0) - jnp.log(1.0 + jnp.exp(-jnp.abs(neg)))
        total = jnp.sum(-alpha * ls_pos) + jnp.sum(-ls_neg)
        o_ref[0, 0] = total * scale

    out = pl.pallas_call(
        body,
        out_shape=jax.ShapeDtypeStruct((1, 1), jnp.float32),
        out_specs=pl.BlockSpec(memory_space=pltpu.SMEM),
    )(dots.reshape(rows, 128), ys.reshape(n_pos // 128, 128))
    return out[0, 0]


def kernel(pos_idxs, ys, neg_idxs, W, num_neg):
    n_pos = pos_idxs.shape[1]
    n_pairs = n_pos + neg_idxs.shape[1]
    u_idx = jnp.concatenate([pos_idxs[0], neg_idxs[0]]).astype(jnp.int32)
    v_idx = jnp.concatenate([pos_idxs[1], neg_idxs[1]]).astype(jnp.int32)
    dots = _sc_dots(u_idx, v_idx, W, n_pairs)
    return _tc_loss(dots.reshape(n_pairs), ys, n_pos, n_pairs)
